# baseline (speedup 1.0000x reference)
import jax, jax.numpy as jnp
def kernel(inputs, weight, feature_table):
    return jnp.concatenate([inputs, inputs[:, :1]], axis=1)


# speedup vs baseline: 1.0081x; 1.0081x over previous
import functools

import jax
import jax.numpy as jnp
from jax import lax
from jax.experimental import pallas as pl
from jax.experimental.pallas import tpu as pltpu

E = 128
V = 128
F = 7
T = 15
SCALE = 1.0 / (E ** 0.5)


def _body(x_ref, w_ref, ft_ref, o_ref, mw_ref):
    i = pl.program_id(0)
    j = pl.program_id(1)

    @pl.when((i == 0) & (j == 0))
    def _compose():
        ft = ft_ref[...]
        t_row = lax.broadcasted_iota(jnp.int32, (V, T), 1)
        m = jnp.zeros((V, T), jnp.float32)
        for jj in range(F):
            m = m + (ft[:, jj:jj + 1] == t_row).astype(jnp.float32)
        row0 = (lax.broadcasted_iota(jnp.int32, (1, T), 1) == 0).astype(jnp.float32)
        pad = jnp.zeros((2 * E - (V + 1), T), jnp.float32)
        m_full = jnp.concatenate([row0, m, pad], axis=0)
        mw_ref[...] = lax.dot_general(m_full, w_ref[...],
                                      (((1,), (0,)), ((), ())),
                                      preferred_element_type=jnp.float32) * SCALE

    mwj = mw_ref[pl.ds(j * E, E), :]
    o_ref[...] = lax.dot_general(x_ref[...], mwj,
                                 (((1,), (1,)), ((), ())),
                                 preferred_element_type=jnp.float32)


@jax.jit
def kernel(inputs, weight, feature_table):
    B = inputs.shape[0]
    blk = 2048
    grid = (B // blk, 2)
    return pl.pallas_call(
        _body,
        grid=grid,
        in_specs=[
            pl.BlockSpec((blk, E), lambda i, j: (i, 0)),
            pl.BlockSpec((T, E), lambda i, j: (0, 0)),
            pl.BlockSpec((V, F), lambda i, j: (0, 0)),
        ],
        out_specs=pl.BlockSpec((blk, E), lambda i, j: (i, j)),
        out_shape=jax.ShapeDtypeStruct((B, V + 1), jnp.float32),
        scratch_shapes=[pltpu.VMEM((2 * E, E), jnp.float32)],
    )(inputs, weight, feature_table)


# speedup vs baseline: 3.5235x; 3.4951x over previous
import jax
import jax.numpy as jnp
from jax import lax
from jax.experimental import pallas as pl
from jax.experimental.pallas import tpu as pltpu

E = 128
V = 128
F = 7
T = 15
SCALE = 1.0 / (E ** 0.5)


def _body(x_ref, w_ref, ft_ref, o_ref, mw_ref):
    @pl.when(pl.program_id(0) == 0)
    def _compose():
        ft = ft_ref[...]
        t_row = lax.broadcasted_iota(jnp.int32, (V, T), 1)
        m = jnp.zeros((V, T), jnp.float32)
        for jj in range(F):
            m = m + (ft[:, jj:jj + 1] == t_row).astype(jnp.float32)
        row0 = (lax.broadcasted_iota(jnp.int32, (1, T), 1) == 0).astype(jnp.float32)
        m_full = jnp.concatenate([row0, m], axis=0)
        mw_ref[...] = lax.dot_general(m_full, w_ref[...],
                                      (((1,), (0,)), ((), ())),
                                      preferred_element_type=jnp.float32) * SCALE

    o_ref[...] = lax.dot_general(mw_ref[...], x_ref[...],
                                 (((1,), (1,)), ((), ())),
                                 preferred_element_type=jnp.float32)


@jax.jit
def kernel(inputs, weight, feature_table):
    B = inputs.shape[0]
    blk = 4096
    grid = (B // blk,)
    ot = pl.pallas_call(
        _body,
        grid=grid,
        in_specs=[
            pl.BlockSpec((blk, E), lambda i: (i, 0)),
            pl.BlockSpec((T, E), lambda i: (0, 0)),
            pl.BlockSpec((V, F), lambda i: (0, 0)),
        ],
        out_specs=pl.BlockSpec((V + 1, blk), lambda i: (0, i)),
        out_shape=jax.ShapeDtypeStruct((V + 1, B), jnp.float32),
        scratch_shapes=[pltpu.VMEM((V + 1, E), jnp.float32)],
    )(inputs, weight, feature_table)
    return ot.T


# speedup vs baseline: 4.0435x; 1.1476x over previous
import jax
import jax.numpy as jnp
from jax import lax
from jax.experimental import pallas as pl
from jax.experimental.pallas import tpu as pltpu

E = 128
V = 128
F = 7
T = 15
SCALE = 1.0 / (E ** 0.5)


def _body(x_ref, w_ref, ft_ref, o_ref, mw_ref):
    @pl.when(pl.program_id(0) == 0)
    def _compose():
        ft = ft_ref[...]
        t_row = lax.broadcasted_iota(jnp.int32, (V, T), 1)
        m = jnp.zeros((V, T), jnp.float32)
        for jj in range(F):
            m = m + (ft[:, jj:jj + 1] == t_row).astype(jnp.float32)
        row0 = (lax.broadcasted_iota(jnp.int32, (1, T), 1) == 0).astype(jnp.float32)
        m_full = jnp.concatenate([row0, m], axis=0)
        mw_ref[...] = lax.dot_general(m_full, w_ref[...],
                                      (((1,), (0,)), ((), ())),
                                      preferred_element_type=jnp.float32) * SCALE

    o_ref[...] = lax.dot_general(mw_ref[...], x_ref[...],
                                 (((1,), (1,)), ((), ())),
                                 preferred_element_type=jnp.float32)


@jax.jit
def kernel(inputs, weight, feature_table):
    B = inputs.shape[0]
    blk = 8192
    grid = (B // blk,)
    ot = pl.pallas_call(
        _body,
        grid=grid,
        in_specs=[
            pl.BlockSpec((blk, E), lambda i: (i, 0)),
            pl.BlockSpec((T, E), lambda i: (0, 0)),
            pl.BlockSpec((V, F), lambda i: (0, 0)),
        ],
        out_specs=pl.BlockSpec((V + 1, blk), lambda i: (0, i)),
        out_shape=jax.ShapeDtypeStruct((V + 1, B), jnp.float32),
        scratch_shapes=[pltpu.VMEM((V + 1, E), jnp.float32)],
    )(inputs, weight, feature_table)
    return ot.T
